# Initial kernel scaffold; baseline (speedup 1.0000x reference)
import jax
import jax.numpy as jnp
from jax.experimental import pallas as pl


def kernel(h, e, norm, edge_index, A_W0, A_b0, A_W1, A_b1, A_W2, A_b2, B_W0, B_b0, B_W1, B_b1, B_W2, B_b2):
    raise NotImplementedError("write your pallas kernel here")


# speedup vs baseline: 2.7182x; 2.7182x over previous
import functools

import jax
import jax.numpy as jnp
from jax import lax
from jax.experimental import pallas as pl
from jax.experimental.pallas import tpu as pltpu
from jax.experimental.pallas import tpu_sc as plsc

N, E, D, H = 10000, 160000, 256, 1024
DH = D // 2
NS = 16
CH = 128
KI = 80
EPS = KI * CH
EPAD = EPS * NS
NA = N + 16
ZR = 632
BLK = 1000


def _mlp_body(h_ref, norm_ref,
              aw0, ab0, aw1, ab1, aw2, ab2,
              bw0, bb0, bw1, bb1, bw2, bb2,
              ah_ref, b0_ref, b1_ref):
    hn = h_ref[...] * norm_ref[...]
    f32 = jnp.float32
    a = jnp.maximum(jnp.dot(hn, aw0[...], preferred_element_type=f32) + ab0[...], 0.0)
    a = jnp.maximum(jnp.dot(a, aw1[...], preferred_element_type=f32) + ab1[...], 0.0)
    ah_ref[...] = jnp.dot(a, aw2[...], preferred_element_type=f32) + ab2[...]
    b = jnp.maximum(jnp.dot(hn, bw0[...], preferred_element_type=f32) + bb0[...], 0.0)
    b = jnp.maximum(jnp.dot(b, bw1[...], preferred_element_type=f32) + bb1[...], 0.0)
    bh = jnp.dot(b, bw2[...], preferred_element_type=f32) + bb2[...]
    b0_ref[...] = bh[:, :DH]
    b1_ref[...] = bh[:, DH:]


def _mlps(h, norm, aw0, ab0, aw1, ab1, aw2, ab2, bw0, bb0, bw1, bb1, bw2, bb2):
    grid = (N // BLK,)
    row_spec = lambda c: pl.BlockSpec((BLK, c), lambda i: (i, 0))
    w_spec = lambda r, c: pl.BlockSpec((r, c), lambda i: (0, 0))
    return pl.pallas_call(
        _mlp_body,
        grid=grid,
        in_specs=[
            row_spec(D), row_spec(1),
            w_spec(D, H), w_spec(1, H), w_spec(H, H), w_spec(1, H), w_spec(H, D), w_spec(1, D),
            w_spec(D, H), w_spec(1, H), w_spec(H, H), w_spec(1, H), w_spec(H, D), w_spec(1, D),
        ],
        out_specs=[row_spec(D), row_spec(DH), row_spec(DH)],
        out_shape=[
            jax.ShapeDtypeStruct((N, D), jnp.float32),
            jax.ShapeDtypeStruct((N, DH), jnp.float32),
            jax.ShapeDtypeStruct((N, DH), jnp.float32),
        ],
    )(h, norm, aw0, ab0, aw1, ab1, aw2, ab2, bw0, bb0, bw1, bb1, bw2, bb2)


def _seg_sum(bh0, bh1, srcm, dstm, zer):
    mesh = plsc.VectorSubcoreMesh(core_axis_name="c", subcore_axis_name="s")

    @functools.partial(
        pl.kernel,
        out_type=[
            jax.ShapeDtypeStruct((N, DH), jnp.float32),
            jax.ShapeDtypeStruct((N, DH), jnp.float32),
        ],
        mesh=mesh,
        scratch_types=[
            pltpu.VMEM_SHARED((NA, DH), jnp.float32),
            pltpu.VMEM((KI, CH), jnp.int32),
            pltpu.VMEM((KI, CH), jnp.int32),
            pltpu.VMEM((CH, DH), jnp.float32),
            pltpu.SemaphoreType.DMA,
        ],
    )
    def seg(bh0_h, bh1_h, srcm_h, dstm_h, zer_h, out0_h, out1_h,
            acc, src_v, dst_v, rows_v, sem):
        cid = lax.axis_index("c")
        sid = lax.axis_index("s")

        def body(bh_h, out_h):
            zbase = pl.multiple_of(jnp.minimum(sid * ZR, NA - ZR), 8)
            obase = pl.multiple_of(jnp.minimum(sid * ZR, N - ZR), 8)
            pltpu.sync_copy(zer_h, acc.at[pl.ds(zbase, ZR)])
            pltpu.sync_copy(srcm_h.at[pl.ds(sid * KI, KI)], src_v)
            pltpu.sync_copy(dstm_h.at[pl.ds(sid * KI, KI)], dst_v)
            plsc.subcore_barrier()

            def step(k, carry):
                pltpu.async_copy(bh_h.at[src_v.at[k]], rows_v, sem).wait()
                pltpu.sync_copy(rows_v, acc.at[dst_v.at[k]], add=True)
                return carry

            lax.fori_loop(0, KI, step, 0)
            plsc.subcore_barrier()
            pltpu.sync_copy(acc.at[pl.ds(obase, ZR)],
                            out_h.at[pl.ds(obase, ZR)])

        pl.when(cid == 0)(lambda: body(bh0_h, out0_h))
        pl.when(cid == 1)(lambda: body(bh1_h, out1_h))

    return seg(bh0, bh1, srcm, dstm, zer)


def _combine_body(ah_ref, a0_ref, a1_ref, norm_ref, out_ref):
    nrm = norm_ref[...]
    out_ref[:, :DH] = (ah_ref[:, :DH] + a0_ref[...]) * nrm
    out_ref[:, DH:] = (ah_ref[:, DH:] + a1_ref[...]) * nrm


def _combine(ah, a0, a1, norm):
    grid = (N // BLK,)
    row_spec = lambda c: pl.BlockSpec((BLK, c), lambda i: (i, 0))
    return pl.pallas_call(
        _combine_body,
        grid=grid,
        in_specs=[row_spec(D), row_spec(DH), row_spec(DH), row_spec(1)],
        out_specs=row_spec(D),
        out_shape=jax.ShapeDtypeStruct((N, D), jnp.float32),
    )(ah, a0, a1, norm)


def kernel(h, e, norm, edge_index,
           A_W0, A_b0, A_W1, A_b1, A_W2, A_b2,
           B_W0, B_b0, B_W1, B_b1, B_W2, B_b2):
    src = edge_index[0].astype(jnp.int32)
    dst = edge_index[1].astype(jnp.int32)
    pad = EPAD - E
    srcm = jnp.concatenate([src, jnp.zeros((pad,), jnp.int32)]).reshape(-1, CH)
    dstm = jnp.concatenate([dst, jnp.full((pad,), N, jnp.int32)]).reshape(-1, CH)
    zer = jnp.zeros((ZR, DH), jnp.float32)

    ah, bh0, bh1 = _mlps(
        h, norm,
        A_W0, A_b0.reshape(1, H), A_W1, A_b1.reshape(1, H), A_W2, A_b2.reshape(1, D),
        B_W0, B_b0.reshape(1, H), B_W1, B_b1.reshape(1, H), B_W2, B_b2.reshape(1, D),
    )
    agg0, agg1 = _seg_sum(bh0, bh1, srcm, dstm, zer)
    h_new = _combine(ah, agg0, agg1, norm)
    return (h_new, e)


# speedup vs baseline: 2.9462x; 1.0839x over previous
import functools

import jax
import jax.numpy as jnp
from jax import lax
from jax.experimental import pallas as pl
from jax.experimental.pallas import tpu as pltpu
from jax.experimental.pallas import tpu_sc as plsc

N, E, D, H = 10000, 160000, 256, 1024
DH = D // 2
NS = 16
CH = 128
KI = 80
EPS = KI * CH
EPAD = EPS * NS
NA = N + 16
ZR = 632
NB = 2
KH = KI // 2
BLK = 1000


def _mlp_body(h_ref, norm_ref,
              aw0, ab0, aw1, ab1, aw2, ab2,
              bw0, bb0, bw1, bb1, bw2, bb2,
              ah_ref, b0_ref, b1_ref):
    hn = h_ref[...] * norm_ref[...]
    f32 = jnp.float32
    a = jnp.maximum(jnp.dot(hn, aw0[...], preferred_element_type=f32) + ab0[...], 0.0)
    a = jnp.maximum(jnp.dot(a, aw1[...], preferred_element_type=f32) + ab1[...], 0.0)
    ah_ref[...] = jnp.dot(a, aw2[...], preferred_element_type=f32) + ab2[...]
    b = jnp.maximum(jnp.dot(hn, bw0[...], preferred_element_type=f32) + bb0[...], 0.0)
    b = jnp.maximum(jnp.dot(b, bw1[...], preferred_element_type=f32) + bb1[...], 0.0)
    bh = jnp.dot(b, bw2[...], preferred_element_type=f32) + bb2[...]
    b0_ref[...] = bh[:, :DH]
    b1_ref[...] = bh[:, DH:]


def _mlps(h, norm, aw0, ab0, aw1, ab1, aw2, ab2, bw0, bb0, bw1, bb1, bw2, bb2):
    grid = (N // BLK,)
    row_spec = lambda c: pl.BlockSpec((BLK, c), lambda i: (i, 0))
    w_spec = lambda r, c: pl.BlockSpec((r, c), lambda i: (0, 0))
    return pl.pallas_call(
        _mlp_body,
        grid=grid,
        in_specs=[
            row_spec(D), row_spec(1),
            w_spec(D, H), w_spec(1, H), w_spec(H, H), w_spec(1, H), w_spec(H, D), w_spec(1, D),
            w_spec(D, H), w_spec(1, H), w_spec(H, H), w_spec(1, H), w_spec(H, D), w_spec(1, D),
        ],
        out_specs=[row_spec(D), row_spec(DH), row_spec(DH)],
        out_shape=[
            jax.ShapeDtypeStruct((N, D), jnp.float32),
            jax.ShapeDtypeStruct((N, DH), jnp.float32),
            jax.ShapeDtypeStruct((N, DH), jnp.float32),
        ],
    )(h, norm, aw0, ab0, aw1, ab1, aw2, ab2, bw0, bb0, bw1, bb1, bw2, bb2)


def _seg_sum(bh0, bh1, srcm, dstm, zer):
    mesh = plsc.VectorSubcoreMesh(core_axis_name="c", subcore_axis_name="s")

    @functools.partial(
        pl.kernel,
        out_type=[
            jax.ShapeDtypeStruct((N, DH), jnp.float32),
            jax.ShapeDtypeStruct((N, DH), jnp.float32),
        ],
        mesh=mesh,
        scratch_types=[
            pltpu.VMEM_SHARED((NA, DH), jnp.float32),
            pltpu.VMEM((KH, CH), jnp.int32),
            pltpu.VMEM((KH, CH), jnp.int32),
            pltpu.VMEM((NB, CH, DH), jnp.float32),
            [pltpu.SemaphoreType.DMA] * NB,
            [pltpu.SemaphoreType.DMA] * NB,
        ],
    )
    def seg(bh0_h, bh1_h, srcm_h, dstm_h, zer_h, out0_h, out1_h,
            acc, src_v, dst_v, rows_v, gsem, ssem):
        cid = lax.axis_index("c")
        sid = lax.axis_index("s")

        def body(bh_h, out_h):
            zbase = pl.multiple_of(jnp.minimum(sid * ZR, NA - ZR), 8)
            obase = pl.multiple_of(jnp.minimum(sid * ZR, N - ZR), 8)
            pltpu.sync_copy(zer_h, acc.at[pl.ds(zbase, ZR)])

            def gather(k, b):
                pltpu.async_copy(bh_h.at[src_v.at[k]], rows_v.at[b], gsem[b])

            def gather_wait(k, b):
                pltpu.make_async_copy(bh_h.at[src_v.at[k]], rows_v.at[b],
                                      gsem[b]).wait()

            def scatter(k, b):
                pltpu.async_copy(rows_v.at[b], acc.at[dst_v.at[k]], ssem[b],
                                 add=True)

            def scatter_wait(k, b):
                pltpu.make_async_copy(rows_v.at[b], acc.at[dst_v.at[k]],
                                      ssem[b]).wait()

            first = True
            for half in range(KI // KH):
                base = sid * KI + half * KH
                pltpu.sync_copy(srcm_h.at[pl.ds(base, KH)], src_v)
                pltpu.sync_copy(dstm_h.at[pl.ds(base, KH)], dst_v)
                if first:
                    plsc.subcore_barrier()
                    first = False
                for b in range(NB):
                    gather(b, b)

                def group(g, carry):
                    for b in range(NB):
                        k = g * NB + b
                        gather_wait(k, b)
                        scatter(k, b)
                    for b in range(NB):
                        kn = g * NB + NB + b
                        scatter_wait(kn - NB, b)
                        gather(kn, b)
                    return carry

                lax.fori_loop(0, KH // NB - 1, group, 0)
                for b in range(NB):
                    k = KH - NB + b
                    gather_wait(k, b)
                    scatter(k, b)
                for b in range(NB):
                    scatter_wait(KH - NB + b, b)
            plsc.subcore_barrier()
            pltpu.sync_copy(acc.at[pl.ds(obase, ZR)],
                            out_h.at[pl.ds(obase, ZR)])

        pl.when(cid == 0)(lambda: body(bh0_h, out0_h))
        pl.when(cid == 1)(lambda: body(bh1_h, out1_h))

    return seg(bh0, bh1, srcm, dstm, zer)


def _combine_body(ah_ref, a0_ref, a1_ref, norm_ref, out_ref):
    nrm = norm_ref[...]
    out_ref[:, :DH] = (ah_ref[:, :DH] + a0_ref[...]) * nrm
    out_ref[:, DH:] = (ah_ref[:, DH:] + a1_ref[...]) * nrm


def _combine(ah, a0, a1, norm):
    grid = (N // BLK,)
    row_spec = lambda c: pl.BlockSpec((BLK, c), lambda i: (i, 0))
    return pl.pallas_call(
        _combine_body,
        grid=grid,
        in_specs=[row_spec(D), row_spec(DH), row_spec(DH), row_spec(1)],
        out_specs=row_spec(D),
        out_shape=jax.ShapeDtypeStruct((N, D), jnp.float32),
    )(ah, a0, a1, norm)


def kernel(h, e, norm, edge_index,
           A_W0, A_b0, A_W1, A_b1, A_W2, A_b2,
           B_W0, B_b0, B_W1, B_b1, B_W2, B_b2):
    src = edge_index[0].astype(jnp.int32)
    dst = edge_index[1].astype(jnp.int32)
    pad = EPAD - E
    srcm = jnp.concatenate([src, jnp.zeros((pad,), jnp.int32)]).reshape(-1, CH)
    dstm = jnp.concatenate([dst, jnp.full((pad,), N, jnp.int32)]).reshape(-1, CH)
    zer = jnp.zeros((ZR, DH), jnp.float32)

    ah, bh0, bh1 = _mlps(
        h, norm,
        A_W0, A_b0.reshape(1, H), A_W1, A_b1.reshape(1, H), A_W2, A_b2.reshape(1, D),
        B_W0, B_b0.reshape(1, H), B_W1, B_b1.reshape(1, H), B_W2, B_b2.reshape(1, D),
    )
    agg0, agg1 = _seg_sum(bh0, bh1, srcm, dstm, zer)
    h_new = _combine(ah, agg0, agg1, norm)
    return (h_new, e)
